# baseline (speedup 1.0000x reference)
import functools

import jax
import jax.numpy as jnp
from jax import lax
from jax.experimental import pallas as pl
from jax.experimental.pallas import tpu as pltpu
from jax.experimental.pallas import tpu_sc as plsc

R, C, D, B = 1000, 8, 128, 16384
GW = 2 * C * D + 128
LOG_OFF = 2 * C * D

NC, NS = 2, 16
NW = NC * NS
BPW = B // NW
CHUNK = 32
NCHUNK = BPW // CHUNK

TBLK = 256


def _sc_gather(idx, tab):
  mesh = plsc.VectorSubcoreMesh(core_axis_name="c", subcore_axis_name="s")

  @functools.partial(
      pl.kernel,
      out_type=jax.ShapeDtypeStruct((B, GW), jnp.float32),
      mesh=mesh,
      scratch_types=[
          pltpu.VMEM((BPW,), jnp.int32),
          pltpu.VMEM((CHUNK, GW), jnp.float32),
          pltpu.SemaphoreType.DMA,
      ],
  )
  def k(idx_hbm, tab_hbm, out_hbm, idx_v, rows_v, sem):
    wid = lax.axis_index("s") * NC + lax.axis_index("c")
    base = wid * BPW
    pltpu.sync_copy(idx_hbm.at[pl.ds(base, BPW)], idx_v)

    def body(ci, carry):
      off = ci * CHUNK
      pltpu.async_copy(
          tab_hbm.at[idx_v.at[pl.ds(off, CHUNK)]], rows_v, sem).wait()
      pltpu.sync_copy(rows_v, out_hbm.at[pl.ds(base + off, CHUNK)])
      return carry

    lax.fori_loop(0, NCHUNK, body, 0)

  return k(idx, tab)


def _tc_moments(g):
  def body(g_ref, mu_ref, lv_ref):
    lo = g_ref[:, LOG_OFF:LOG_OFF + 128]
    m = jnp.max(lo, axis=-1, keepdims=True)
    e = jnp.exp(lo - m)
    w = e / jnp.sum(e, axis=-1, keepdims=True)
    mu_acc = jnp.zeros((TBLK, D), jnp.float32)
    sm_acc = jnp.zeros((TBLK, D), jnp.float32)
    for c in range(C):
      wc = w[:, c:c + 1]
      mu_c = g_ref[:, c * D:(c + 1) * D]
      lv_c = g_ref[:, C * D + c * D:C * D + (c + 1) * D]
      mu_acc = mu_acc + wc * mu_c
      sm_acc = sm_acc + wc * (jnp.exp(lv_c) + mu_c * mu_c)
    var = jnp.clip(sm_acc - mu_acc * mu_acc, 1e-6, None)
    mu_ref[...] = mu_acc
    lv_ref[...] = jnp.log(var)

  return pl.pallas_call(
      body,
      grid=(B // TBLK,),
      in_specs=[pl.BlockSpec((TBLK, GW), lambda i: (i, 0))],
      out_specs=[
          pl.BlockSpec((TBLK, D), lambda i: (i, 0)),
          pl.BlockSpec((TBLK, D), lambda i: (i, 0)),
      ],
      out_shape=[
          jax.ShapeDtypeStruct((B, D), jnp.float32),
          jax.ShapeDtypeStruct((B, D), jnp.float32),
      ],
  )(g)


def kernel(regime_id, regime_seen_mask, logits_emb, mu_emb, logvar_emb,
           logits_unknown, mu_unknown, logvar_unknown):
  rid = jnp.clip(regime_id, 0, R - 1).astype(jnp.int32)
  idx = jnp.where(regime_seen_mask, rid, R).astype(jnp.int32)
  pad = jnp.full((R + 1, GW - LOG_OFF - C), -1e30, jnp.float32)
  tab = jnp.concatenate([
      jnp.concatenate([mu_emb, mu_unknown.reshape(1, C * D)], 0),
      jnp.concatenate([logvar_emb, logvar_unknown.reshape(1, C * D)], 0),
      jnp.concatenate([logits_emb, logits_unknown.reshape(1, C)], 0),
      pad,
  ], axis=1)
  g = _sc_gather(idx, tab)
  mu, logvar = _tc_moments(g)
  return (mu, logvar)


# speedup vs baseline: 1.1905x; 1.1905x over previous
import functools

import jax
import jax.numpy as jnp
from jax import lax
from jax.experimental import pallas as pl
from jax.experimental.pallas import tpu as pltpu
from jax.experimental.pallas import tpu_sc as plsc

R, C, D, B = 1000, 8, 128, 16384
GW = 2 * C * D + 16
LOG_OFF = 2 * C * D

NC, NS, L = 2, 16, 16
NW = NC * NS
BPW = B // NW
NE = 16
NCH = BPW // NE
DG = D // L

_LN2 = 0.6931471805599453


def _vlog(x):
  bits = plsc.bitcast(x, jnp.int32)
  e = jnp.right_shift(bits, 23) - 127
  m = plsc.bitcast(
      jnp.bitwise_or(jnp.bitwise_and(bits, 0x007FFFFF), 0x3F800000),
      jnp.float32)
  big = m > 1.4142135623730951
  m = jnp.where(big, m * 0.5, m)
  e = jnp.where(big, e + 1, e)
  s = (m - 1.0) / (m + 1.0)
  s2 = s * s
  p = 2.0 * s * (1.0 + s2 * (1.0 / 3.0 + s2 * (0.2 + s2 * (1.0 / 7.0))))
  return e.astype(jnp.float32) * _LN2 + p


def _sc_fused(idx, tab):
  mesh = plsc.VectorSubcoreMesh(core_axis_name="c", subcore_axis_name="s")

  @functools.partial(
      pl.kernel,
      out_type=[
          jax.ShapeDtypeStruct((B, D), jnp.float32),
          jax.ShapeDtypeStruct((B, D), jnp.float32),
      ],
      mesh=mesh,
      compiler_params=pltpu.CompilerParams(use_tc_tiling_on_sc=False,
                                           needs_layout_passes=False),
      scratch_types=[
          pltpu.VMEM((BPW,), jnp.int32),
          pltpu.VMEM((NE, GW), jnp.float32),
          pltpu.VMEM((NE, GW), jnp.float32),
          pltpu.VMEM((NE, D), jnp.float32),
          pltpu.VMEM((NE, D), jnp.float32),
          pltpu.SemaphoreType.DMA,
          pltpu.SemaphoreType.DMA,
      ],
  )
  def k(idx_hbm, tab_hbm, mu_hbm, lv_hbm,
        idx_v, buf0, buf1, omu, olv, sem0, sem1):
    wid = lax.axis_index("s") * NC + lax.axis_index("c")
    base = wid * BPW
    pltpu.sync_copy(idx_hbm.at[pl.ds(base, BPW)], idx_v)

    bufs = (buf0, buf1)
    sems = (sem0, sem1)

    def start_gather(ci, slot):
      pltpu.async_copy(
          tab_hbm.at[idx_v.at[pl.ds(ci * NE, NE)]], bufs[slot], sems[slot])

    def wait_gather(slot):
      pltpu.make_async_copy(
          tab_hbm.at[idx_v.at[pl.ds(0, NE)]], bufs[slot], sems[slot]).wait()

    rows = lax.iota(jnp.int32, L)

    def compute(ci, buf):
      ls = [plsc.load_gather(buf, [rows, jnp.full((L,), LOG_OFF + c,
                                                  jnp.int32)])
            for c in range(C)]
      mx = ls[0]
      for c in range(1, C):
        mx = jnp.maximum(mx, ls[c])
      es = [jnp.exp(l - mx) for l in ls]
      tot = es[0]
      for c in range(1, C):
        tot = tot + es[c]
      inv = 1.0 / tot
      ws = [e * inv for e in es]

      def dstep(dd, carry):
        mu_acc = jnp.zeros((L,), jnp.float32)
        sm_acc = jnp.zeros((L,), jnp.float32)
        for c in range(C):
          mu_cd = plsc.load_gather(
              buf, [rows, jnp.full((L,), c * D, jnp.int32) + dd])
          lv_cd = plsc.load_gather(
              buf, [rows, jnp.full((L,), C * D + c * D, jnp.int32) + dd])
          mu_acc = mu_acc + ws[c] * mu_cd
          sm_acc = sm_acc + ws[c] * (jnp.exp(lv_cd) + mu_cd * mu_cd)
        var = jnp.maximum(sm_acc - mu_acc * mu_acc, 1e-6)
        dvec = jnp.full((L,), dd, jnp.int32)
        plsc.store_scatter(omu, [rows, dvec], mu_acc)
        plsc.store_scatter(olv, [rows, dvec], _vlog(var))
        return carry

      lax.fori_loop(0, D, dstep, 0)
      pltpu.sync_copy(omu, mu_hbm.at[pl.ds(base + ci * NE, NE)])
      pltpu.sync_copy(olv, lv_hbm.at[pl.ds(base + ci * NE, NE)])

    start_gather(0, 0)
    start_gather(1, 1)

    def outer(oi, carry):
      for b in range(2):
        ci = oi * 2 + b
        wait_gather(b)
        compute(ci, bufs[b])

        @pl.when(ci + 2 < NCH)
        def _():
          start_gather(ci + 2, b)
      return carry

    lax.fori_loop(0, NCH // 2, outer, 0)

  return k(idx, tab)


def kernel(regime_id, regime_seen_mask, logits_emb, mu_emb, logvar_emb,
           logits_unknown, mu_unknown, logvar_unknown):
  rid = jnp.clip(regime_id, 0, R - 1).astype(jnp.int32)
  idx = jnp.where(regime_seen_mask, rid, R).astype(jnp.int32)
  pad = jnp.full((R + 1, GW - LOG_OFF - C), -1e30, jnp.float32)
  tab = jnp.concatenate([
      jnp.concatenate([mu_emb, mu_unknown.reshape(1, C * D)], 0),
      jnp.concatenate([logvar_emb, logvar_unknown.reshape(1, C * D)], 0),
      jnp.concatenate([logits_emb, logits_unknown.reshape(1, C)], 0),
      pad,
  ], axis=1)
  mu, logvar = _sc_fused(idx, tab)
  return (mu, logvar)


# speedup vs baseline: 1.5430x; 1.2960x over previous
import functools

import jax
import jax.numpy as jnp
from jax import lax
from jax.experimental import pallas as pl
from jax.experimental.pallas import tpu as pltpu
from jax.experimental.pallas import tpu_sc as plsc

R, C, D, B = 1000, 8, 128, 16384
GW = 2 * C * D + 16
LOG_OFF = 2 * C * D

NC, NS, L = 2, 16, 16
NW = NC * NS
BPW = B // NW
NE = 16
NCH = BPW // NE
DG = D // L

_LN2 = 0.6931471805599453


def _vlog(x):
  bits = plsc.bitcast(x, jnp.int32)
  e = jnp.right_shift(bits, 23) - 127
  m = plsc.bitcast(
      jnp.bitwise_or(jnp.bitwise_and(bits, 0x007FFFFF), 0x3F800000),
      jnp.float32)
  big = m > 1.4142135623730951
  m = jnp.where(big, m * 0.5, m)
  e = jnp.where(big, e + 1, e)
  s = (m - 1.0) / (m + 1.0)
  s2 = s * s
  p = 2.0 * s * (1.0 + s2 * (1.0 / 3.0 + s2 * (0.2 + s2 * (1.0 / 7.0))))
  return e.astype(jnp.float32) * _LN2 + p


def _sc_fused(idx, tab):
  mesh = plsc.VectorSubcoreMesh(core_axis_name="c", subcore_axis_name="s")

  @functools.partial(
      pl.kernel,
      out_type=[
          jax.ShapeDtypeStruct((B, D), jnp.float32),
          jax.ShapeDtypeStruct((B, D), jnp.float32),
      ],
      mesh=mesh,
      compiler_params=pltpu.CompilerParams(use_tc_tiling_on_sc=False,
                                           needs_layout_passes=False),
      scratch_types=[
          pltpu.VMEM((BPW,), jnp.int32),
          pltpu.VMEM((NE, GW), jnp.float32),
          pltpu.VMEM((NE, GW), jnp.float32),
          pltpu.VMEM((NE, D), jnp.float32),
          pltpu.VMEM((NE, D), jnp.float32),
          pltpu.SemaphoreType.DMA,
          pltpu.SemaphoreType.DMA,
      ],
  )
  def k(idx_hbm, tab_hbm, mu_hbm, lv_hbm,
        idx_v, buf0, buf1, omu, olv, sem0, sem1):
    wid = lax.axis_index("s") * NC + lax.axis_index("c")
    base = wid * BPW
    pltpu.sync_copy(idx_hbm.at[pl.ds(base, BPW)], idx_v)

    bufs = (buf0, buf1)
    sems = (sem0, sem1)

    def start_gather(ci, slot):
      pltpu.async_copy(
          tab_hbm.at[idx_v.at[pl.ds(ci * NE, NE)]], bufs[slot], sems[slot])

    def wait_gather(slot):
      pltpu.make_async_copy(
          tab_hbm.at[idx_v.at[pl.ds(0, NE)]], bufs[slot], sems[slot]).wait()

    rows = lax.iota(jnp.int32, L)

    def compute(ci, buf):
      ls = [plsc.load_gather(buf, [rows, jnp.full((L,), LOG_OFF + c,
                                                  jnp.int32)])
            for c in range(C)]
      mx = ls[0]
      for c in range(1, C):
        mx = jnp.maximum(mx, ls[c])
      es = [jnp.exp(l - mx) for l in ls]
      tot = es[0]
      for c in range(1, C):
        tot = tot + es[c]
      inv = 1.0 / tot
      ws = [e * inv for e in es]

      def dstep(dd, carry):
        dvec = jnp.bitwise_and(rows + dd, D - 1)
        mu_acc = jnp.zeros((L,), jnp.float32)
        sm_acc = jnp.zeros((L,), jnp.float32)
        for c in range(C):
          mu_cd = plsc.load_gather(
              buf, [rows, jnp.full((L,), c * D, jnp.int32) + dvec])
          lv_cd = plsc.load_gather(
              buf, [rows, jnp.full((L,), C * D + c * D, jnp.int32) + dvec])
          mu_acc = mu_acc + ws[c] * mu_cd
          sm_acc = sm_acc + ws[c] * (jnp.exp(lv_cd) + mu_cd * mu_cd)
        var = jnp.maximum(sm_acc - mu_acc * mu_acc, 1e-6)
        plsc.store_scatter(omu, [rows, dvec], mu_acc)
        plsc.store_scatter(olv, [rows, dvec], _vlog(var))
        return carry

      lax.fori_loop(0, D, dstep, 0)
      pltpu.sync_copy(omu, mu_hbm.at[pl.ds(base + ci * NE, NE)])
      pltpu.sync_copy(olv, lv_hbm.at[pl.ds(base + ci * NE, NE)])

    start_gather(0, 0)
    start_gather(1, 1)

    def outer(oi, carry):
      for b in range(2):
        ci = oi * 2 + b
        wait_gather(b)
        compute(ci, bufs[b])

        @pl.when(ci + 2 < NCH)
        def _():
          start_gather(ci + 2, b)
      return carry

    lax.fori_loop(0, NCH // 2, outer, 0)

  return k(idx, tab)


def kernel(regime_id, regime_seen_mask, logits_emb, mu_emb, logvar_emb,
           logits_unknown, mu_unknown, logvar_unknown):
  rid = jnp.clip(regime_id, 0, R - 1).astype(jnp.int32)
  idx = jnp.where(regime_seen_mask, rid, R).astype(jnp.int32)
  pad = jnp.full((R + 1, GW - LOG_OFF - C), -1e30, jnp.float32)
  tab = jnp.concatenate([
      jnp.concatenate([mu_emb, mu_unknown.reshape(1, C * D)], 0),
      jnp.concatenate([logvar_emb, logvar_unknown.reshape(1, C * D)], 0),
      jnp.concatenate([logits_emb, logits_unknown.reshape(1, C)], 0),
      pad,
  ], axis=1)
  mu, logvar = _sc_fused(idx, tab)
  return (mu, logvar)
